# baseline (speedup 1.0000x reference)
import jax
import jax.numpy as jnp
from jax import lax
from jax.experimental import pallas as pl
from jax.experimental.pallas import tpu as pltpu

NCLS = 21
ALPHA = 10.0


def _loss_kernel(cats_ref, bbs_ref, gtb_ref, gt_ref, anc_ref,
                 conf_ref, loc_ref, n_ref, total_ref):
    b = pl.program_id(0)
    nb = pl.num_programs(0)

    x = cats_ref[0]
    gt = gt_ref[0]
    maskf = (gt != NCLS - 1).astype(jnp.float32)
    cls = lax.broadcasted_iota(jnp.int32, x.shape, 1)
    z = (cls == gt).astype(jnp.float32)
    bce = jnp.maximum(x, 0.0) - x * z + jnp.log1p(jnp.exp(-jnp.abs(x)))
    w = maskf * (cls < NCLS - 1).astype(jnp.float32)
    conf_p = jnp.sum(bce * w)

    diff = anc_ref[...] + bbs_ref[0] - jnp.clip(gtb_ref[0], 0.0, 1.0)
    ad = jnp.abs(diff)
    sl1 = jnp.where(ad < 1.0, 0.5 * diff * diff, ad - 0.5)
    loc_p = jnp.sum(sl1 * maskf)
    n_p = jnp.sum(maskf)

    @pl.when(b == 0)
    def _init():
        conf_ref[0, 0] = 0.0
        loc_ref[0, 0] = 0.0
        n_ref[0, 0] = 0.0

    conf_ref[0, 0] += conf_p
    loc_ref[0, 0] += loc_p
    n_ref[0, 0] += n_p

    @pl.when(b == nb - 1)
    def _fin():
        total_ref[0, 0] = (conf_ref[0, 0] + ALPHA * loc_ref[0, 0]) / n_ref[0, 0]


def kernel(bbs_preds, cats_preds, gt_bbs, gt_cats, anchors):
    batch, n_anchors, _ = cats_preds.shape
    gt3 = gt_cats.astype(jnp.int32).reshape(batch, n_anchors, 1)

    smem_scalar = pl.BlockSpec((1, 1), lambda b: (0, 0), memory_space=pltpu.SMEM)
    conf, loc, n, total = pl.pallas_call(
        _loss_kernel,
        grid=(batch,),
        in_specs=[
            pl.BlockSpec((1, n_anchors, NCLS), lambda b: (b, 0, 0)),
            pl.BlockSpec((1, n_anchors, 4), lambda b: (b, 0, 0)),
            pl.BlockSpec((1, n_anchors, 4), lambda b: (b, 0, 0)),
            pl.BlockSpec((1, n_anchors, 1), lambda b: (b, 0, 0)),
            pl.BlockSpec((n_anchors, 4), lambda b: (0, 0)),
        ],
        out_specs=[smem_scalar] * 4,
        out_shape=[jax.ShapeDtypeStruct((1, 1), jnp.float32)] * 4,
        compiler_params=pltpu.CompilerParams(
            dimension_semantics=("arbitrary",)),
    )(cats_preds, bbs_preds, gt_bbs, gt3, anchors)
    return (total[0, 0], loc[0, 0], conf[0, 0])


# speedup vs baseline: 6.3977x; 6.3977x over previous
import jax
import jax.numpy as jnp
from jax import lax
from jax.experimental import pallas as pl
from jax.experimental.pallas import tpu as pltpu

NCLS = 21
ALPHA = 10.0


def _loss_kernel(cats_ref, bbs_ref, gtb_ref, gt_ref, anc_ref,
                 conf_ref, loc_ref, n_ref, total_ref):
    b = pl.program_id(0)
    nb = pl.num_programs(0)

    x = cats_ref[0]
    gt = gt_ref[0]
    maskf = (gt != NCLS - 1).astype(jnp.float32)

    xs = x[:NCLS - 1]
    sp = jnp.maximum(xs, 0.0) + jnp.log1p(jnp.exp(-jnp.abs(xs)))
    spsum = jnp.sum(sp, axis=0, keepdims=True)
    cls = lax.broadcasted_iota(jnp.int32, x.shape, 0)
    xz = jnp.sum(jnp.where(cls == gt, x, 0.0), axis=0, keepdims=True)
    conf_p = jnp.sum((spsum - xz) * maskf)

    d = anc_ref[...] + bbs_ref[0] - jnp.clip(gtb_ref[0], 0.0, 1.0)
    ad = jnp.abs(d)
    sl1 = jnp.where(ad < 1.0, 0.5 * d * d, ad - 0.5)
    loc_p = jnp.sum(jnp.sum(sl1, axis=0, keepdims=True) * maskf)
    n_p = jnp.sum(maskf)

    @pl.when(b == 0)
    def _init():
        conf_ref[0, 0] = 0.0
        loc_ref[0, 0] = 0.0
        n_ref[0, 0] = 0.0

    conf_ref[0, 0] += conf_p
    loc_ref[0, 0] += loc_p
    n_ref[0, 0] += n_p

    @pl.when(b == nb - 1)
    def _fin():
        total_ref[0, 0] = (conf_ref[0, 0] + ALPHA * loc_ref[0, 0]) / n_ref[0, 0]


def kernel(bbs_preds, cats_preds, gt_bbs, gt_cats, anchors):
    batch, n_anchors, _ = cats_preds.shape
    cats_t = jnp.transpose(cats_preds, (0, 2, 1))
    bbs_t = jnp.transpose(bbs_preds, (0, 2, 1))
    gtb_t = jnp.transpose(gt_bbs, (0, 2, 1))
    anc_t = anchors.T
    gt3 = gt_cats.astype(jnp.int32).reshape(batch, 1, n_anchors)

    smem_scalar = pl.BlockSpec((1, 1), lambda b: (0, 0), memory_space=pltpu.SMEM)
    conf, loc, n, total = pl.pallas_call(
        _loss_kernel,
        grid=(batch,),
        in_specs=[
            pl.BlockSpec((1, NCLS, n_anchors), lambda b: (b, 0, 0)),
            pl.BlockSpec((1, 4, n_anchors), lambda b: (b, 0, 0)),
            pl.BlockSpec((1, 4, n_anchors), lambda b: (b, 0, 0)),
            pl.BlockSpec((1, 1, n_anchors), lambda b: (b, 0, 0)),
            pl.BlockSpec((4, n_anchors), lambda b: (0, 0)),
        ],
        out_specs=[smem_scalar] * 4,
        out_shape=[jax.ShapeDtypeStruct((1, 1), jnp.float32)] * 4,
        compiler_params=pltpu.CompilerParams(
            dimension_semantics=("arbitrary",)),
    )(cats_t, bbs_t, gtb_t, gt3, anc_t)
    return (total[0, 0], loc[0, 0], conf[0, 0])


# speedup vs baseline: 12.9350x; 2.0218x over previous
import jax
import jax.numpy as jnp
from jax.experimental import pallas as pl
from jax.experimental.pallas import tpu as pltpu

NCLS = 21
ALPHA = 10.0
NBOX = 4


def _loss_kernel(cats_ref, bbs_ref, gtb_ref, anc_ref, gt_ref,
                 conf_ref, loc_ref, n_ref, total_ref, mask_ref):
    s = pl.program_id(0)
    nb = pl.num_programs(0)

    @pl.when(s == 0)
    def _init():
        maskf = (gt_ref[...] != NCLS - 1).astype(jnp.float32)
        mask_ref[...] = maskf
        conf_ref[0, 0] = 0.0
        loc_ref[0, 0] = 0.0
        n_ref[0, 0] = jnp.sum(maskf)

    maskf = mask_ref[...]

    @pl.when(s < NBOX)
    def _box():
        d = anc_ref[0] + bbs_ref[0] - jnp.clip(gtb_ref[0], 0.0, 1.0)
        ad = jnp.abs(d)
        sl1 = jnp.where(ad < 1.0, 0.5 * d * d, ad - 0.5)
        loc_ref[0, 0] += jnp.sum(sl1 * maskf)

    @pl.when(s >= NBOX)
    def _cls():
        c = s - NBOX
        x = cats_ref[0]
        sp = jnp.maximum(x, 0.0) + jnp.log1p(jnp.exp(-jnp.abs(x)))
        xz = jnp.where(gt_ref[...] == c, x, 0.0)
        conf_ref[0, 0] += jnp.sum((sp - xz) * maskf)

    @pl.when(s == nb - 1)
    def _fin():
        total_ref[0, 0] = (conf_ref[0, 0] + ALPHA * loc_ref[0, 0]) / n_ref[0, 0]


def kernel(bbs_preds, cats_preds, gt_bbs, gt_cats, anchors):
    batch, n_anchors, _ = cats_preds.shape
    cats_t = jnp.transpose(cats_preds, (2, 0, 1))
    bbs_t = jnp.transpose(bbs_preds, (2, 0, 1))
    gtb_t = jnp.transpose(gt_bbs, (2, 0, 1))
    anc_t = anchors.T.reshape(NBOX, 1, n_anchors)
    gt = gt_cats.astype(jnp.int32)

    grid = (NBOX + NCLS - 1,)
    conf, loc, n, total = pl.pallas_call(
        _loss_kernel,
        grid=grid,
        in_specs=[
            pl.BlockSpec((1, batch, n_anchors),
                         lambda s: (jnp.maximum(s - NBOX, 0), 0, 0)),
            pl.BlockSpec((1, batch, n_anchors),
                         lambda s: (jnp.minimum(s, NBOX - 1), 0, 0)),
            pl.BlockSpec((1, batch, n_anchors),
                         lambda s: (jnp.minimum(s, NBOX - 1), 0, 0)),
            pl.BlockSpec((1, 1, n_anchors),
                         lambda s: (jnp.minimum(s, NBOX - 1), 0, 0)),
            pl.BlockSpec((batch, n_anchors), lambda s: (0, 0)),
        ],
        out_specs=[pl.BlockSpec((1, 1), lambda s: (0, 0),
                                memory_space=pltpu.SMEM)] * 4,
        out_shape=[jax.ShapeDtypeStruct((1, 1), jnp.float32)] * 4,
        scratch_shapes=[pltpu.VMEM((batch, n_anchors), jnp.float32)],
        compiler_params=pltpu.CompilerParams(
            dimension_semantics=("arbitrary",)),
    )(cats_t, bbs_t, gtb_t, anc_t, gt)
    return (total[0, 0], loc[0, 0], conf[0, 0])


# speedup vs baseline: 14.7337x; 1.1391x over previous
import jax
import jax.numpy as jnp
from jax.experimental import pallas as pl
from jax.experimental.pallas import tpu as pltpu

NCLS = 21
ALPHA = 10.0
NBOX = 4
LOG2E = 1.4426950408889634
LN2 = 0.6931471805599453


def _loss_kernel(cats_ref, bbs_ref, gtb_ref, anc_ref, gt_ref,
                 conf_ref, loc_ref, n_ref, total_ref, aconf_ref, abox_ref):
    s = pl.program_id(0)
    nb = pl.num_programs(0)

    @pl.when(s < NBOX)
    def _box():
        d = anc_ref[0] + bbs_ref[0] - jnp.clip(gtb_ref[0], 0.0, 1.0)
        ad = jnp.abs(d)
        sl1 = jnp.where(ad < 1.0, 0.5 * d * d, ad - 0.5)

        @pl.when(s == 0)
        def _first():
            abox_ref[...] = sl1

        @pl.when(s > 0)
        def _rest():
            abox_ref[...] += sl1

    @pl.when(s >= NBOX)
    def _cls():
        c = s - NBOX
        x = cats_ref[0]
        e2 = jax.lax.exp2(jnp.minimum(x * LOG2E, 86.0))
        sp = LN2 * jnp.log2(1.0 + e2)
        term = sp - jnp.where(gt_ref[...] == c, x, 0.0)

        @pl.when(s == NBOX)
        def _first():
            aconf_ref[...] = term

        @pl.when(s > NBOX)
        def _rest():
            aconf_ref[...] += term

    @pl.when(s == nb - 1)
    def _fin():
        maskf = (gt_ref[...] != NCLS - 1).astype(jnp.float32)
        conf = jnp.sum(aconf_ref[...] * maskf)
        loc = jnp.sum(abox_ref[...] * maskf)
        n = jnp.sum(maskf)
        conf_ref[0, 0] = conf
        loc_ref[0, 0] = loc
        n_ref[0, 0] = n
        total_ref[0, 0] = (conf + ALPHA * loc) / n


def kernel(bbs_preds, cats_preds, gt_bbs, gt_cats, anchors):
    batch, n_anchors, _ = cats_preds.shape
    cats_t = jnp.transpose(cats_preds, (2, 0, 1))
    bbs_t = jnp.transpose(bbs_preds, (2, 0, 1))
    gtb_t = jnp.transpose(gt_bbs, (2, 0, 1))
    anc_t = anchors.T.reshape(NBOX, 1, n_anchors)
    gt = gt_cats.astype(jnp.int32)

    grid = (NBOX + NCLS - 1,)
    conf, loc, n, total = pl.pallas_call(
        _loss_kernel,
        grid=grid,
        in_specs=[
            pl.BlockSpec((1, batch, n_anchors),
                         lambda s: (jnp.maximum(s - NBOX, 0), 0, 0)),
            pl.BlockSpec((1, batch, n_anchors),
                         lambda s: (jnp.minimum(s, NBOX - 1), 0, 0)),
            pl.BlockSpec((1, batch, n_anchors),
                         lambda s: (jnp.minimum(s, NBOX - 1), 0, 0)),
            pl.BlockSpec((1, 1, n_anchors),
                         lambda s: (jnp.minimum(s, NBOX - 1), 0, 0)),
            pl.BlockSpec((batch, n_anchors), lambda s: (0, 0)),
        ],
        out_specs=[pl.BlockSpec((1, 1), lambda s: (0, 0),
                                memory_space=pltpu.SMEM)] * 4,
        out_shape=[jax.ShapeDtypeStruct((1, 1), jnp.float32)] * 4,
        scratch_shapes=[pltpu.VMEM((batch, n_anchors), jnp.float32),
                        pltpu.VMEM((batch, n_anchors), jnp.float32)],
        compiler_params=pltpu.CompilerParams(
            dimension_semantics=("arbitrary",)),
    )(cats_t, bbs_t, gtb_t, anc_t, gt)
    return (total[0, 0], loc[0, 0], conf[0, 0])


# speedup vs baseline: 15.7776x; 1.0708x over previous
import jax
import jax.numpy as jnp
from jax.experimental import pallas as pl
from jax.experimental.pallas import tpu as pltpu

NCLS = 21
ALPHA = 10.0
NBOX = 4
CPB = 2
LOG2E = 1.4426950408889634
LN2 = 0.6931471805599453


def _loss_kernel(cats_ref, bbs_ref, gtb_ref, anc_ref, gt_ref,
                 conf_ref, loc_ref, n_ref, total_ref, aconf_ref, abox_ref):
    s = pl.program_id(0)
    nb = pl.num_programs(0)

    @pl.when(s == 0)
    def _box():
        acc = None
        for c in range(NBOX):
            d = anc_ref[c] + bbs_ref[c] - jnp.clip(gtb_ref[c], 0.0, 1.0)
            ad = jnp.abs(d)
            sl1 = jnp.where(ad < 1.0, 0.5 * d * d, ad - 0.5)
            acc = sl1 if acc is None else acc + sl1
        abox_ref[...] = acc

    @pl.when(s > 0)
    def _cls():
        gt = gt_ref[...]
        acc = None
        for k in range(CPB):
            c = (s - 1) * CPB + k
            x = cats_ref[k]
            e2 = jax.lax.exp2(jnp.minimum(x * LOG2E, 100.0))
            sp = LN2 * jnp.log2(1.0 + e2)
            term = sp - jnp.where(gt == c, x, 0.0)
            acc = term if acc is None else acc + term

        @pl.when(s == 1)
        def _first():
            aconf_ref[...] = acc

        @pl.when(s > 1)
        def _rest():
            aconf_ref[...] += acc

    @pl.when(s == nb - 1)
    def _fin():
        maskf = (gt_ref[...] != NCLS - 1).astype(jnp.float32)
        conf = jnp.sum(aconf_ref[...] * maskf)
        loc = jnp.sum(abox_ref[...] * maskf)
        n = jnp.sum(maskf)
        conf_ref[0, 0] = conf
        loc_ref[0, 0] = loc
        n_ref[0, 0] = n
        total_ref[0, 0] = (conf + ALPHA * loc) / n


def kernel(bbs_preds, cats_preds, gt_bbs, gt_cats, anchors):
    batch, n_anchors, _ = cats_preds.shape
    cats_t = jnp.transpose(cats_preds, (2, 0, 1))
    bbs_t = jnp.transpose(bbs_preds, (2, 0, 1))
    gtb_t = jnp.transpose(gt_bbs, (2, 0, 1))
    anc_t = anchors.T.reshape(NBOX, 1, n_anchors)
    gt = gt_cats.astype(jnp.int32)

    grid = (1 + (NCLS - 1) // CPB,)
    conf, loc, n, total = pl.pallas_call(
        _loss_kernel,
        grid=grid,
        in_specs=[
            pl.BlockSpec((CPB, batch, n_anchors),
                         lambda s: (jnp.maximum(s - 1, 0), 0, 0)),
            pl.BlockSpec((NBOX, batch, n_anchors), lambda s: (0, 0, 0)),
            pl.BlockSpec((NBOX, batch, n_anchors), lambda s: (0, 0, 0)),
            pl.BlockSpec((NBOX, 1, n_anchors), lambda s: (0, 0, 0)),
            pl.BlockSpec((batch, n_anchors), lambda s: (0, 0)),
        ],
        out_specs=[pl.BlockSpec((1, 1), lambda s: (0, 0),
                                memory_space=pltpu.SMEM)] * 4,
        out_shape=[jax.ShapeDtypeStruct((1, 1), jnp.float32)] * 4,
        scratch_shapes=[pltpu.VMEM((batch, n_anchors), jnp.float32),
                        pltpu.VMEM((batch, n_anchors), jnp.float32)],
        compiler_params=pltpu.CompilerParams(
            dimension_semantics=("arbitrary",)),
    )(cats_t, bbs_t, gtb_t, anc_t, gt)
    return (total[0, 0], loc[0, 0], conf[0, 0])
